# baseline (speedup 1.0000x reference)
import numpy as np

import jax
import jax.numpy as jnp
from jax import lax
from jax.experimental import pallas as pl
from jax.experimental.pallas import tpu as pltpu
from jax.experimental.pallas import tpu_sc as plsc

N_ROWS = 16384
D = 64
C = 64
MARGIN_ = 2.0

NUM_CORES = 2
NUM_SUBCORES = 16
NW = NUM_CORES * NUM_SUBCORES
RPW = N_ROWS // NW
L = 16
LPW = RPW // 2

NORM_BLK = 512


def _norm_one(f):
    sq = jnp.sum(f * f, axis=1, keepdims=True)
    inv = 1.0 / jnp.maximum(jnp.sqrt(sq), 1e-12)
    return f * inv, jnp.sum(sq * (inv * inv))


def _norm_body(fa_ref, fb_ref, fn_ref, sq_ref):
    fna, sqa = _norm_one(fa_ref[...])
    fnb, sqb = _norm_one(fb_ref[...])
    fn_ref[...] = jnp.concatenate([fna, fnb], axis=1)
    sq_ref[...] = jnp.broadcast_to(sqa + sqb, (8, 128))


_GRID_A = N_ROWS // NORM_BLK // 2


_normalize = pl.pallas_call(
    _norm_body,
    grid=(_GRID_A,),
    in_specs=[
        pl.BlockSpec((NORM_BLK, D), lambda i: (i, 0)),
        pl.BlockSpec((NORM_BLK, D), lambda i: (i + _GRID_A, 0)),
    ],
    out_specs=[
        pl.BlockSpec((NORM_BLK, 2 * D), lambda i: (i, 0)),
        pl.BlockSpec((8, 128), lambda i: (i, 0)),
    ],
    out_shape=[
        jax.ShapeDtypeStruct((N_ROWS // 2, 2 * D), jnp.float32),
        jax.ShapeDtypeStruct((8 * _GRID_A, 128), jnp.float32),
    ],
)


def _sc_body(fn2, labels, sums_out, cnt_out, fvm, lvm, acc, cnt2):
    cid = lax.axis_index("c")
    sid = lax.axis_index("s")
    wid = sid * NUM_CORES + cid

    pltpu.sync_copy(fn2.at[pl.ds(wid * LPW, LPW)], fvm)
    pltpu.sync_copy(labels.at[pl.ds(wid * LPW, LPW)], lvm.at[pl.ds(0, LPW)])
    pltpu.sync_copy(
        labels.at[pl.ds(N_ROWS // 2 + wid * LPW, LPW)], lvm.at[pl.ds(LPW, LPW)]
    )

    zeros = jnp.zeros((L,), jnp.float32)
    ones = jnp.ones((L,), jnp.float32)

    def zero_body(k, carry):
        for j in range(D // L):
            acc[k, pl.ds(L * j, L)] = zeros
        cnt2[k, pl.ds(0, L)] = zeros
        return carry

    lax.fori_loop(0, C, zero_body, 0)

    def row_body(g, carry):
        lab_a = lvm[pl.ds(g * L, L)]
        lab_b = lvm[pl.ds(LPW + g * L, L)]
        for u in range(L):
            line = g * L + u
            la = lab_a[u]
            lb = lab_b[u]
            for j in range(D // L):
                plsc.addupdate(
                    acc.at[la, pl.ds(L * j, L)], fvm[line, pl.ds(L * j, L)]
                )
            plsc.addupdate(cnt2.at[la], ones)
            for j in range(D // L):
                plsc.addupdate(
                    acc.at[lb, pl.ds(L * j, L)], fvm[line, pl.ds(D + L * j, L)]
                )
            plsc.addupdate(cnt2.at[lb], ones)
        return carry

    lax.fori_loop(0, LPW // L, row_body, 0)

    pltpu.sync_copy(acc, sums_out.at[wid])
    pltpu.sync_copy(cnt2, cnt_out.at[wid])


_sc_segment = pl.kernel(
    _sc_body,
    out_type=[
        jax.ShapeDtypeStruct((NW, C, D), jnp.float32),
        jax.ShapeDtypeStruct((NW, C, L), jnp.float32),
    ],
    mesh=plsc.VectorSubcoreMesh(
        core_axis_name="c", subcore_axis_name="s",
        num_cores=NUM_CORES, num_subcores=NUM_SUBCORES,
    ),
    scratch_types=[
        pltpu.VMEM((LPW, 2 * D), jnp.float32),
        pltpu.VMEM((RPW,), jnp.int32),
        pltpu.VMEM((C, D), jnp.float32),
        pltpu.VMEM((C, L), jnp.float32),
    ],
    compiler_params=pltpu.CompilerParams(needs_layout_passes=False),
)


def _epi_body(sums_ref, cnt_ref, sq_ref, out_ref):
    sums = jnp.sum(sums_ref[...], axis=0)
    counts = jnp.sum(cnt_ref[...], axis=0)[:, 0]
    sqtot = jnp.sum(sq_ref[...]) / (8.0 * 128.0)
    csafe = jnp.maximum(counts, 1.0)
    centers = sums / csafe[:, None]
    cnorm2 = jnp.sum(centers * centers, axis=1)
    intra = (sqtot - jnp.sum(counts * cnorm2)) / jnp.float32(N_ROWS)
    gram = jnp.dot(centers, centers.T, preferred_element_type=jnp.float32)
    d2 = cnorm2[:, None] + cnorm2[None, :] - 2.0 * gram
    d2 = jnp.maximum(d2, 0.0)
    row = lax.broadcasted_iota(jnp.int32, (C, C), 0)
    col = lax.broadcasted_iota(jnp.int32, (C, C), 1)
    pres = counts > 0.5
    mask = (row != col) & pres[:, None] & pres[None, :]
    min_d2 = jnp.min(jnp.where(mask, d2, jnp.float32(1e30)))
    min_inter = jnp.sqrt(min_d2)
    inter = jnp.maximum(MARGIN_ - min_inter, 0.0)
    sr = jnp.clip(min_inter / MARGIN_, 0.0, 1.0)
    loss = (1.0 + 2.0 * (1.0 - sr)) * intra + (2.0 * sr) * inter
    npres = jnp.sum(pres.astype(jnp.float32))
    loss = jnp.where(npres < 1.5, jnp.float32(0.0), loss)
    out_ref[...] = jnp.broadcast_to(loss, (1, 1))


_epilogue = pl.pallas_call(
    _epi_body,
    out_shape=jax.ShapeDtypeStruct((1, 1), jnp.float32),
)


@jax.jit
def kernel(features, labels):
    fn2, sqpart = _normalize(features, features)
    sums, cnt = _sc_segment(fn2, labels)
    return _epilogue(sums, cnt, sqpart)[0, 0]


# speedup vs baseline: 1.0232x; 1.0232x over previous
import numpy as np

import jax
import jax.numpy as jnp
from jax import lax
from jax.experimental import pallas as pl
from jax.experimental.pallas import tpu as pltpu
from jax.experimental.pallas import tpu_sc as plsc

N_ROWS = 16384
D = 64
C = 64
MARGIN_ = 2.0

NUM_CORES = 2
NUM_SUBCORES = 16
NW = NUM_CORES * NUM_SUBCORES
RPW = N_ROWS // NW
L = 16
LPW = RPW // 2

NORM_BLK = 512


def _norm_body(f_ref, fn_ref, sq_ref):
    f = f_ref[...]
    sq = jnp.sum(f * f, axis=1, keepdims=True)
    inv = lax.rsqrt(jnp.maximum(sq, 1e-24))
    fn = f * inv
    fn_ref[...] = jnp.concatenate([fn[:NORM_BLK], fn[NORM_BLK:]], axis=1)
    sq_ref[...] = jnp.broadcast_to(jnp.sum(sq * (inv * inv)), (8, 128))


_GRID_A = N_ROWS // (2 * NORM_BLK)


_normalize = pl.pallas_call(
    _norm_body,
    grid=(_GRID_A,),
    in_specs=[pl.BlockSpec((2 * NORM_BLK, D), lambda i: (i, 0))],
    out_specs=[
        pl.BlockSpec((NORM_BLK, 2 * D), lambda i: (i, 0)),
        pl.BlockSpec((8, 128), lambda i: (i, 0)),
    ],
    out_shape=[
        jax.ShapeDtypeStruct((N_ROWS // 2, 2 * D), jnp.float32),
        jax.ShapeDtypeStruct((8 * _GRID_A, 128), jnp.float32),
    ],
)


def _sc_body(fn2, labels, sums_out, cnt_out, fvm, lvm, acc, cnt2):
    cid = lax.axis_index("c")
    sid = lax.axis_index("s")
    wid = sid * NUM_CORES + cid

    pltpu.sync_copy(fn2.at[pl.ds(wid * LPW, LPW)], fvm)
    base_a = (wid // 2) * (4 * LPW) + (wid % 2) * LPW
    base_b = base_a + 2 * LPW
    pltpu.sync_copy(labels.at[pl.ds(base_a, LPW)], lvm.at[pl.ds(0, LPW)])
    pltpu.sync_copy(labels.at[pl.ds(base_b, LPW)], lvm.at[pl.ds(LPW, LPW)])

    zeros = jnp.zeros((L,), jnp.float32)
    ones = jnp.ones((L,), jnp.float32)

    def zero_body(k, carry):
        for j in range(D // L):
            acc[k, pl.ds(L * j, L)] = zeros
        cnt2[k, pl.ds(0, L)] = zeros
        return carry

    lax.fori_loop(0, C, zero_body, 0)

    def row_body(g, carry):
        lab_a = lvm[pl.ds(g * L, L)]
        lab_b = lvm[pl.ds(LPW + g * L, L)]
        for u in range(L):
            line = g * L + u
            la = lab_a[u]
            lb = lab_b[u]
            for j in range(D // L):
                plsc.addupdate(
                    acc.at[la, pl.ds(L * j, L)], fvm[line, pl.ds(L * j, L)]
                )
            plsc.addupdate(cnt2.at[la], ones)
            for j in range(D // L):
                plsc.addupdate(
                    acc.at[lb, pl.ds(L * j, L)], fvm[line, pl.ds(D + L * j, L)]
                )
            plsc.addupdate(cnt2.at[lb], ones)
        return carry

    lax.fori_loop(0, LPW // L, row_body, 0)

    pltpu.sync_copy(acc, sums_out.at[wid])
    pltpu.sync_copy(cnt2, cnt_out.at[wid])


_sc_segment = pl.kernel(
    _sc_body,
    out_type=[
        jax.ShapeDtypeStruct((NW, C, D), jnp.float32),
        jax.ShapeDtypeStruct((NW, C, L), jnp.float32),
    ],
    mesh=plsc.VectorSubcoreMesh(
        core_axis_name="c", subcore_axis_name="s",
        num_cores=NUM_CORES, num_subcores=NUM_SUBCORES,
    ),
    scratch_types=[
        pltpu.VMEM((LPW, 2 * D), jnp.float32),
        pltpu.VMEM((RPW,), jnp.int32),
        pltpu.VMEM((C, D), jnp.float32),
        pltpu.VMEM((C, L), jnp.float32),
    ],
    compiler_params=pltpu.CompilerParams(needs_layout_passes=False),
)


def _epi_body(sums_ref, cnt_ref, sq_ref, out_ref):
    sums = jnp.sum(sums_ref[...], axis=0)
    counts = jnp.sum(cnt_ref[...], axis=0)[:, 0]
    sqtot = jnp.sum(sq_ref[...]) / (8.0 * 128.0)
    csafe = jnp.maximum(counts, 1.0)
    centers = sums / csafe[:, None]
    cnorm2 = jnp.sum(centers * centers, axis=1)
    intra = (sqtot - jnp.sum(counts * cnorm2)) / jnp.float32(N_ROWS)
    gram = jnp.dot(centers, centers.T, preferred_element_type=jnp.float32)
    d2 = cnorm2[:, None] + cnorm2[None, :] - 2.0 * gram
    d2 = jnp.maximum(d2, 0.0)
    row = lax.broadcasted_iota(jnp.int32, (C, C), 0)
    col = lax.broadcasted_iota(jnp.int32, (C, C), 1)
    pres = counts > 0.5
    mask = (row != col) & pres[:, None] & pres[None, :]
    min_d2 = jnp.min(jnp.where(mask, d2, jnp.float32(1e30)))
    min_inter = jnp.sqrt(min_d2)
    inter = jnp.maximum(MARGIN_ - min_inter, 0.0)
    sr = jnp.clip(min_inter / MARGIN_, 0.0, 1.0)
    loss = (1.0 + 2.0 * (1.0 - sr)) * intra + (2.0 * sr) * inter
    npres = jnp.sum(pres.astype(jnp.float32))
    loss = jnp.where(npres < 1.5, jnp.float32(0.0), loss)
    out_ref[...] = jnp.broadcast_to(loss, (1, 1))


_epilogue = pl.pallas_call(
    _epi_body,
    out_shape=jax.ShapeDtypeStruct((1, 1), jnp.float32),
)


@jax.jit
def kernel(features, labels):
    fn2, sqpart = _normalize(features)
    sums, cnt = _sc_segment(fn2, labels)
    return _epilogue(sums, cnt, sqpart)[0, 0]
